# Initial kernel scaffold; baseline (speedup 1.0000x reference)
import jax
import jax.numpy as jnp
from jax.experimental import pallas as pl


def kernel(x, edge_index, Wl1, Wr1, b1, Wl2, Wr2, b2, Wc, bc):
    raise NotImplementedError("write your pallas kernel here")


# speedup vs baseline: 4.5461x; 4.5461x over previous
import functools

import jax
import jax.numpy as jnp
from jax import lax
from jax.experimental import pallas as pl
from jax.experimental.pallas import tpu as pltpu
from jax.experimental.pallas import tpu_sc as plsc

N = 10000
E = 320000
D_IN = 128
D_H = 256
N_CLASSES = 64

NC = 2
NS = 16
DH = 64
C = 80
PER_TILE = E // NS
N_CHUNKS = PER_TILE // C
SLAB = 640
SLAB_LAST = N - SLAB * (NS - 1)
ZR = 40


def _make_sc_agg(n_phases, with_deg):
    mesh = plsc.VectorSubcoreMesh(core_axis_name="c", subcore_axis_name="s",
                                  num_cores=NC, num_subcores=NS)
    n_t = NC * n_phases
    out_type = [jax.ShapeDtypeStruct((n_t, N, DH), jnp.float32)]
    if with_deg:
        out_type.append(jax.ShapeDtypeStruct((N, 16), jnp.float32))
    scratch = [
        pltpu.VMEM((N_CHUNKS, C), jnp.int32),
        pltpu.VMEM((N_CHUNKS, C), jnp.int32),
        pltpu.VMEM((C, DH), jnp.float32),
        pltpu.VMEM((ZR, DH), jnp.float32),
        pltpu.VMEM_SHARED((N, DH), jnp.float32),
        pltpu.SemaphoreType.DMA,
    ]
    if with_deg:
        scratch += [
            pltpu.VMEM((C, 16), jnp.float32),
            pltpu.VMEM((ZR, 16), jnp.float32),
            pltpu.VMEM_SHARED((N, 16), jnp.float32),
        ]

    def body(*args):
        tables = args[:n_t]
        src_hbm, dst_hbm, out_hbm = args[n_t:n_t + 3]
        rest = args[n_t + 3:]
        if with_deg:
            (deg_hbm, idx_s, idx_d, rows, zbuf, accum, sem,
             ones, zbuf16, dacc) = rest
        else:
            idx_s, idx_d, rows, zbuf, accum, sem = rest
        c = lax.axis_index("c")
        s = lax.axis_index("s")
        row0 = s * SLAB

        def for_slab(fn):
            @pl.when(s < NS - 1)
            def _():
                fn(SLAB)

            @pl.when(s == NS - 1)
            def _():
                fn(SLAB_LAST)

        @pl.loop(0, ZR)
        def _(i):
            for k in range(DH // 16):
                zbuf[i, pl.ds(k * 16, 16)] = jnp.zeros((16,), jnp.float32)

        if with_deg:
            @pl.when(c == 0)
            def _():
                @pl.loop(0, C)
                def _(i):
                    ones[i, :] = jnp.full((16,), 1.0, jnp.float32)

                @pl.loop(0, ZR)
                def _(i):
                    zbuf16[i, :] = jnp.zeros((16,), jnp.float32)

                def zero_deg(nr):
                    @pl.loop(0, nr // ZR)
                    def _(i):
                        pltpu.sync_copy(zbuf16,
                                        dacc.at[pl.ds(row0 + i * ZR, ZR)])

                for_slab(zero_deg)

        pltpu.sync_copy(src_hbm.at[s], idx_s)
        pltpu.sync_copy(dst_hbm.at[s], idx_d)

        for p in range(n_phases):
            def zero_accum(nr):
                @pl.loop(0, nr // ZR)
                def _(i):
                    pltpu.sync_copy(zbuf, accum.at[pl.ds(row0 + i * ZR, ZR)])

            for_slab(zero_accum)
            plsc.subcore_barrier()

            def run(table, do_deg):
                @pl.loop(0, N_CHUNKS)
                def _(j):
                    pltpu.async_copy(table.at[idx_s.at[j]], rows, sem).wait()
                    pltpu.sync_copy(rows, accum.at[idx_d.at[j]], add=True)
                    if do_deg:
                        pltpu.sync_copy(ones, dacc.at[idx_d.at[j]], add=True)

            @pl.when(c == 0)
            def _():
                run(tables[2 * p], with_deg and p == 0)

            @pl.when(c == 1)
            def _():
                run(tables[2 * p + 1], False)

            plsc.subcore_barrier()

            def copy_out(nr):
                pltpu.sync_copy(accum.at[pl.ds(row0, nr)],
                                out_hbm.at[2 * p + c, pl.ds(row0, nr)])
                if with_deg and p == 0:
                    @pl.when(c == 0)
                    def _():
                        pltpu.sync_copy(dacc.at[pl.ds(row0, nr)],
                                        deg_hbm.at[pl.ds(row0, nr)])

            for_slab(copy_out)
            if p + 1 < n_phases:
                plsc.subcore_barrier()

    return pl.kernel(body, out_type=tuple(out_type), mesh=mesh,
                     scratch_types=scratch,
                     compiler_params=pltpu.CompilerParams(
                         use_tc_tiling_on_sc=False))


_sc_agg_l1 = functools.cache(lambda: _make_sc_agg(1, with_deg=True))
_sc_agg_l2 = functools.cache(lambda: _make_sc_agg(2, with_deg=False))

R = 1000


def _tc1_body(s_ref, deg_ref, x_ref, wl_ref, wr_ref, b_ref, o_ref):
    deg = jnp.maximum(deg_ref[:, 0:1], 1.0)
    agg = jnp.concatenate([s_ref[0], s_ref[1]], axis=1) / deg
    h = jnp.dot(agg, wl_ref[...], preferred_element_type=jnp.float32)
    h = h + jnp.dot(x_ref[...], wr_ref[...], preferred_element_type=jnp.float32)
    h = jnp.maximum(h + b_ref[...], 0.0)
    for q in range(4):
        o_ref[q] = h[:, q * DH:(q + 1) * DH]


def _tc2_body(s_ref, deg_ref, hp_ref, wl_ref, wr_ref, b_ref, wc_ref, bc_ref,
              o_ref):
    deg = jnp.maximum(deg_ref[:, 0:1], 1.0)
    agg = jnp.concatenate([s_ref[i] for i in range(4)], axis=1) / deg
    h1 = jnp.concatenate([hp_ref[i] for i in range(4)], axis=1)
    z = jnp.dot(agg, wl_ref[...], preferred_element_type=jnp.float32)
    z = z + jnp.dot(h1, wr_ref[...], preferred_element_type=jnp.float32)
    z = z + b_ref[...]
    o_ref[...] = (jnp.dot(z, wc_ref[...], preferred_element_type=jnp.float32)
                  + bc_ref[...])


_tc1 = pl.pallas_call(
    _tc1_body,
    grid=(N // R,),
    in_specs=[
        pl.BlockSpec((NC, R, DH), lambda i: (0, i, 0)),
        pl.BlockSpec((R, 16), lambda i: (i, 0)),
        pl.BlockSpec((R, D_IN), lambda i: (i, 0)),
        pl.BlockSpec((D_IN, D_H), lambda i: (0, 0)),
        pl.BlockSpec((D_IN, D_H), lambda i: (0, 0)),
        pl.BlockSpec((1, D_H), lambda i: (0, 0)),
    ],
    out_specs=pl.BlockSpec((4, R, DH), lambda i: (0, i, 0)),
    out_shape=jax.ShapeDtypeStruct((4, N, DH), jnp.float32),
)

_tc2 = pl.pallas_call(
    _tc2_body,
    grid=(N // R,),
    in_specs=[
        pl.BlockSpec((4, R, DH), lambda i: (0, i, 0)),
        pl.BlockSpec((R, 16), lambda i: (i, 0)),
        pl.BlockSpec((4, R, DH), lambda i: (0, i, 0)),
        pl.BlockSpec((D_H, D_H), lambda i: (0, 0)),
        pl.BlockSpec((D_H, D_H), lambda i: (0, 0)),
        pl.BlockSpec((1, D_H), lambda i: (0, 0)),
        pl.BlockSpec((D_H, N_CLASSES), lambda i: (0, 0)),
        pl.BlockSpec((1, N_CLASSES), lambda i: (0, 0)),
    ],
    out_specs=pl.BlockSpec((R, N_CLASSES), lambda i: (i, 0)),
    out_shape=jax.ShapeDtypeStruct((N, N_CLASSES), jnp.float32),
)


def kernel(x, edge_index, Wl1, Wr1, b1, Wl2, Wr2, b2, Wc, bc):
    src = edge_index[0].astype(jnp.int32).reshape(NS, N_CHUNKS, C)
    dst = edge_index[1].astype(jnp.int32).reshape(NS, N_CHUNKS, C)
    x_lo = x[:, :DH]
    x_hi = x[:, DH:]
    summed1, deg16 = _sc_agg_l1()(x_lo, x_hi, src, dst)
    h1q = _tc1(summed1, deg16, x, Wl1, Wr1, b1.reshape(1, -1))
    (summed2,) = _sc_agg_l2()(h1q[0], h1q[1], h1q[2], h1q[3], src, dst)
    out = _tc2(summed2, deg16, h1q, Wl2, Wr2, b2.reshape(1, -1), Wc,
               bc.reshape(1, -1))
    return out


# speedup vs baseline: 7.5776x; 1.6668x over previous
import functools

import jax
import jax.numpy as jnp
from jax import lax
from jax.experimental import pallas as pl
from jax.experimental.pallas import tpu as pltpu
from jax.experimental.pallas import tpu_sc as plsc

N = 10000
E = 320000
D_IN = 128
D_H = 256
N_CLASSES = 64

NC = 2
NS = 16
DH = 64
C = 80
PER_TILE = E // NS
N_CHUNKS = PER_TILE // C
SLAB = 640
SLAB_LAST = N - SLAB * (NS - 1)
ZR = 40
NBUF = 2


def _make_sc_agg(n_phases, with_deg):
    mesh = plsc.VectorSubcoreMesh(core_axis_name="c", subcore_axis_name="s",
                                  num_cores=NC, num_subcores=NS)
    n_t = NC * n_phases
    out_type = [jax.ShapeDtypeStruct((n_t, N, DH), jnp.float32)]
    if with_deg:
        out_type.append(jax.ShapeDtypeStruct((N, 16), jnp.float32))
    scratch = [
        pltpu.VMEM((N_CHUNKS, C), jnp.int32),
        pltpu.VMEM((N_CHUNKS, C), jnp.int32),
        pltpu.VMEM((NBUF, C, DH), jnp.float32),
        pltpu.VMEM((ZR, DH), jnp.float32),
        pltpu.VMEM_SHARED((N, DH), jnp.float32),
        pltpu.SemaphoreType.DMA((NBUF,)),
    ]
    if with_deg:
        scratch += [
            pltpu.VMEM((C, 16), jnp.float32),
            pltpu.VMEM((ZR, 16), jnp.float32),
            pltpu.VMEM_SHARED((N, 16), jnp.float32),
        ]

    def body(*args):
        tables = args[:n_t]
        src_hbm, dst_hbm, out_hbm = args[n_t:n_t + 3]
        rest = args[n_t + 3:]
        if with_deg:
            (deg_hbm, idx_s, idx_d, rows, zbuf, accum, sem,
             ones, zbuf16, dacc) = rest
        else:
            idx_s, idx_d, rows, zbuf, accum, sem = rest
        c = lax.axis_index("c")
        s = lax.axis_index("s")
        row0 = s * SLAB

        def for_slab(fn):
            @pl.when(s < NS - 1)
            def _():
                fn(SLAB)

            @pl.when(s == NS - 1)
            def _():
                fn(SLAB_LAST)

        @pl.loop(0, ZR)
        def _(i):
            for k in range(DH // 16):
                zbuf[i, pl.ds(k * 16, 16)] = jnp.zeros((16,), jnp.float32)

        if with_deg:
            @pl.when(c == 0)
            def _():
                @pl.loop(0, C)
                def _(i):
                    ones[i, :] = jnp.full((16,), 1.0, jnp.float32)

                @pl.loop(0, ZR)
                def _(i):
                    zbuf16[i, :] = jnp.zeros((16,), jnp.float32)

                def zero_deg(nr):
                    @pl.loop(0, nr // ZR)
                    def _(i):
                        pltpu.sync_copy(zbuf16,
                                        dacc.at[pl.ds(row0 + i * ZR, ZR)])

                for_slab(zero_deg)

        pltpu.sync_copy(src_hbm.at[s], idx_s)
        pltpu.sync_copy(dst_hbm.at[s], idx_d)

        for p in range(n_phases):
            def zero_accum(nr):
                @pl.loop(0, nr // ZR)
                def _(i):
                    pltpu.sync_copy(zbuf, accum.at[pl.ds(row0 + i * ZR, ZR)])

            for_slab(zero_accum)
            plsc.subcore_barrier()

            def run(table, do_deg):
                for b in range(NBUF):
                    pltpu.async_copy(table.at[idx_s.at[b]], rows.at[b],
                                     sem.at[b])

                @pl.loop(0, N_CHUNKS, step=NBUF)
                def _(j):
                    for b in range(NBUF):
                        jj = j + b
                        pltpu.make_async_copy(table.at[idx_s.at[jj]],
                                              rows.at[b], sem.at[b]).wait()
                        pltpu.sync_copy(rows.at[b], accum.at[idx_d.at[jj]],
                                        add=True)
                        if do_deg:
                            pltpu.sync_copy(ones, dacc.at[idx_d.at[jj]],
                                            add=True)
                        @pl.when(jj + NBUF < N_CHUNKS)
                        def _():
                            pltpu.async_copy(
                                table.at[idx_s.at[jj + NBUF]], rows.at[b],
                                sem.at[b])

            @pl.when(c == 0)
            def _():
                run(tables[2 * p], with_deg and p == 0)

            @pl.when(c == 1)
            def _():
                run(tables[2 * p + 1], False)

            plsc.subcore_barrier()

            def copy_out(nr):
                pltpu.sync_copy(accum.at[pl.ds(row0, nr)],
                                out_hbm.at[2 * p + c, pl.ds(row0, nr)])
                if with_deg and p == 0:
                    @pl.when(c == 0)
                    def _():
                        pltpu.sync_copy(dacc.at[pl.ds(row0, nr)],
                                        deg_hbm.at[pl.ds(row0, nr)])

            for_slab(copy_out)
            if p + 1 < n_phases:
                plsc.subcore_barrier()

    return pl.kernel(body, out_type=tuple(out_type), mesh=mesh,
                     scratch_types=scratch,
                     compiler_params=pltpu.CompilerParams(
                         use_tc_tiling_on_sc=False))


_sc_agg_l1 = functools.cache(lambda: _make_sc_agg(1, with_deg=True))
_sc_agg_l2 = functools.cache(lambda: _make_sc_agg(2, with_deg=False))

R = 1000


def _tc1_body(s_ref, deg_ref, x_ref, wl_ref, wr_ref, b_ref, o_ref):
    deg = jnp.maximum(deg_ref[:, 0:1], 1.0)
    agg = jnp.concatenate([s_ref[0], s_ref[1]], axis=1) / deg
    h = jnp.dot(agg, wl_ref[...], preferred_element_type=jnp.float32)
    h = h + jnp.dot(x_ref[...], wr_ref[...], preferred_element_type=jnp.float32)
    h = jnp.maximum(h + b_ref[...], 0.0)
    for q in range(4):
        o_ref[q] = h[:, q * DH:(q + 1) * DH]


def _tc2_body(s_ref, deg_ref, hp_ref, wl_ref, wr_ref, b_ref, wc_ref, bc_ref,
              o_ref):
    deg = jnp.maximum(deg_ref[:, 0:1], 1.0)
    agg = jnp.concatenate([s_ref[i] for i in range(4)], axis=1) / deg
    h1 = jnp.concatenate([hp_ref[i] for i in range(4)], axis=1)
    z = jnp.dot(agg, wl_ref[...], preferred_element_type=jnp.float32)
    z = z + jnp.dot(h1, wr_ref[...], preferred_element_type=jnp.float32)
    z = z + b_ref[...]
    o_ref[...] = (jnp.dot(z, wc_ref[...], preferred_element_type=jnp.float32)
                  + bc_ref[...])


_tc1 = pl.pallas_call(
    _tc1_body,
    grid=(N // R,),
    in_specs=[
        pl.BlockSpec((NC, R, DH), lambda i: (0, i, 0)),
        pl.BlockSpec((R, 16), lambda i: (i, 0)),
        pl.BlockSpec((R, D_IN), lambda i: (i, 0)),
        pl.BlockSpec((D_IN, D_H), lambda i: (0, 0)),
        pl.BlockSpec((D_IN, D_H), lambda i: (0, 0)),
        pl.BlockSpec((1, D_H), lambda i: (0, 0)),
    ],
    out_specs=pl.BlockSpec((4, R, DH), lambda i: (0, i, 0)),
    out_shape=jax.ShapeDtypeStruct((4, N, DH), jnp.float32),
)

_tc2 = pl.pallas_call(
    _tc2_body,
    grid=(N // R,),
    in_specs=[
        pl.BlockSpec((4, R, DH), lambda i: (0, i, 0)),
        pl.BlockSpec((R, 16), lambda i: (i, 0)),
        pl.BlockSpec((4, R, DH), lambda i: (0, i, 0)),
        pl.BlockSpec((D_H, D_H), lambda i: (0, 0)),
        pl.BlockSpec((D_H, D_H), lambda i: (0, 0)),
        pl.BlockSpec((1, D_H), lambda i: (0, 0)),
        pl.BlockSpec((D_H, N_CLASSES), lambda i: (0, 0)),
        pl.BlockSpec((1, N_CLASSES), lambda i: (0, 0)),
    ],
    out_specs=pl.BlockSpec((R, N_CLASSES), lambda i: (i, 0)),
    out_shape=jax.ShapeDtypeStruct((N, N_CLASSES), jnp.float32),
)


def kernel(x, edge_index, Wl1, Wr1, b1, Wl2, Wr2, b2, Wc, bc):
    src = edge_index[0].astype(jnp.int32).reshape(NS, N_CHUNKS, C)
    dst = edge_index[1].astype(jnp.int32).reshape(NS, N_CHUNKS, C)
    x_lo = x[:, :DH]
    x_hi = x[:, DH:]
    summed1, deg16 = _sc_agg_l1()(x_lo, x_hi, src, dst)
    h1q = _tc1(summed1, deg16, x, Wl1, Wr1, b1.reshape(1, -1))
    (summed2,) = _sc_agg_l2()(h1q[0], h1q[1], h1q[2], h1q[3], src, dst)
    out = _tc2(summed2, deg16, h1q, Wl2, Wr2, b2.reshape(1, -1), Wc,
               bc.reshape(1, -1))
    return out


# speedup vs baseline: 10.8917x; 1.4374x over previous
import functools

import jax
import jax.numpy as jnp
from jax import lax
from jax.experimental import pallas as pl
from jax.experimental.pallas import tpu as pltpu
from jax.experimental.pallas import tpu_sc as plsc

N = 10000
E = 320000
D_IN = 128
D_H = 256
N_CLASSES = 64

NC = 2
NS = 16
DH = 64
C = 80
PER_TILE = E // NS
N_CHUNKS = PER_TILE // C
SLAB = 640
SLAB_LAST = N - SLAB * (NS - 1)
ZR = 40
NBUF = 5


def _make_sc_agg(n_phases, with_deg):
    mesh = plsc.VectorSubcoreMesh(core_axis_name="c", subcore_axis_name="s",
                                  num_cores=NC, num_subcores=NS)
    n_t = NC * n_phases
    out_type = [jax.ShapeDtypeStruct((n_t, N, DH), jnp.float32)]
    if with_deg:
        out_type.append(jax.ShapeDtypeStruct((N, 16), jnp.float32))
    scratch = [
        pltpu.VMEM((N_CHUNKS, C), jnp.int32),
        pltpu.VMEM((N_CHUNKS, C), jnp.int32),
        pltpu.VMEM((NBUF, C, DH), jnp.float32),
        pltpu.VMEM((ZR, DH), jnp.float32),
        pltpu.VMEM_SHARED((N, DH), jnp.float32),
        pltpu.SemaphoreType.DMA((NBUF,)),
        pltpu.SemaphoreType.DMA((NBUF,)),
    ]
    if with_deg:
        scratch += [
            pltpu.VMEM((C, 16), jnp.float32),
            pltpu.VMEM((ZR, 16), jnp.float32),
            pltpu.VMEM_SHARED((N, 16), jnp.float32),
        ]

    def body(*args):
        tables = args[:n_t]
        src_hbm, dst_hbm, out_hbm = args[n_t:n_t + 3]
        rest = args[n_t + 3:]
        if with_deg:
            (deg_hbm, idx_s, idx_d, rows, zbuf, accum, gsem, ssem,
             ones, zbuf16, dacc) = rest
        else:
            idx_s, idx_d, rows, zbuf, accum, gsem, ssem = rest
        c = lax.axis_index("c")
        s = lax.axis_index("s")
        row0 = s * SLAB

        def for_slab(fn):
            @pl.when(s < NS - 1)
            def _():
                fn(SLAB)

            @pl.when(s == NS - 1)
            def _():
                fn(SLAB_LAST)

        @pl.loop(0, ZR)
        def _(i):
            for k in range(DH // 16):
                zbuf[i, pl.ds(k * 16, 16)] = jnp.zeros((16,), jnp.float32)

        if with_deg:
            @pl.when(c == 0)
            def _():
                @pl.loop(0, C)
                def _(i):
                    ones[i, :] = jnp.full((16,), 1.0, jnp.float32)

                @pl.loop(0, ZR)
                def _(i):
                    zbuf16[i, :] = jnp.zeros((16,), jnp.float32)

                def zero_deg(nr):
                    @pl.loop(0, nr // ZR)
                    def _(i):
                        pltpu.sync_copy(zbuf16,
                                        dacc.at[pl.ds(row0 + i * ZR, ZR)])

                for_slab(zero_deg)

        pltpu.sync_copy(src_hbm.at[s], idx_s)
        pltpu.sync_copy(dst_hbm.at[s], idx_d)

        for p in range(n_phases):
            def zero_accum(nr):
                @pl.loop(0, nr // ZR)
                def _(i):
                    pltpu.sync_copy(zbuf, accum.at[pl.ds(row0 + i * ZR, ZR)])

            for_slab(zero_accum)
            plsc.subcore_barrier()

            def run(table, do_deg):
                for b in range(NBUF):
                    pltpu.async_copy(table.at[idx_s.at[b]], rows.at[b],
                                     gsem.at[b])

                @pl.loop(0, N_CHUNKS, step=NBUF)
                def _(j):
                    for b in range(NBUF):
                        jj = j + b
                        pltpu.make_async_copy(table.at[idx_s.at[jj]],
                                              rows.at[b], gsem.at[b]).wait()
                        pltpu.async_copy(rows.at[b], accum.at[idx_d.at[jj]],
                                         ssem.at[b], add=True)
                        if do_deg:
                            pltpu.sync_copy(ones, dacc.at[idx_d.at[jj]],
                                            add=True)
                        @pl.when(jj + NBUF < N_CHUNKS)
                        def _():
                            pltpu.make_async_copy(
                                rows.at[b], accum.at[idx_d.at[jj]],
                                ssem.at[b]).wait()
                            pltpu.async_copy(
                                table.at[idx_s.at[jj + NBUF]], rows.at[b],
                                gsem.at[b])

                for b in range(NBUF):
                    jj = N_CHUNKS - NBUF + b
                    pltpu.make_async_copy(rows.at[b], accum.at[idx_d.at[jj]],
                                          ssem.at[b]).wait()

            @pl.when(c == 0)
            def _():
                run(tables[2 * p], with_deg and p == 0)

            @pl.when(c == 1)
            def _():
                run(tables[2 * p + 1], False)

            plsc.subcore_barrier()

            def copy_out(nr):
                pltpu.sync_copy(accum.at[pl.ds(row0, nr)],
                                out_hbm.at[2 * p + c, pl.ds(row0, nr)])
                if with_deg and p == 0:
                    @pl.when(c == 0)
                    def _():
                        pltpu.sync_copy(dacc.at[pl.ds(row0, nr)],
                                        deg_hbm.at[pl.ds(row0, nr)])

            for_slab(copy_out)
            if p + 1 < n_phases:
                plsc.subcore_barrier()

    return pl.kernel(body, out_type=tuple(out_type), mesh=mesh,
                     scratch_types=scratch,
                     compiler_params=pltpu.CompilerParams(
                         use_tc_tiling_on_sc=False))


_sc_agg_l1 = functools.cache(lambda: _make_sc_agg(1, with_deg=True))
_sc_agg_l2 = functools.cache(lambda: _make_sc_agg(2, with_deg=False))

R = 1000


def _tc1_body(s_ref, deg_ref, x_ref, wl_ref, wr_ref, b_ref, o_ref):
    deg = jnp.maximum(deg_ref[:, 0:1], 1.0)
    agg = jnp.concatenate([s_ref[0], s_ref[1]], axis=1) / deg
    h = jnp.dot(agg, wl_ref[...], preferred_element_type=jnp.float32)
    h = h + jnp.dot(x_ref[...], wr_ref[...], preferred_element_type=jnp.float32)
    h = jnp.maximum(h + b_ref[...], 0.0)
    for q in range(4):
        o_ref[q] = h[:, q * DH:(q + 1) * DH]


def _tc2_body(s_ref, deg_ref, hp_ref, wl_ref, wr_ref, b_ref, wc_ref, bc_ref,
              o_ref):
    deg = jnp.maximum(deg_ref[:, 0:1], 1.0)
    agg = jnp.concatenate([s_ref[i] for i in range(4)], axis=1) / deg
    h1 = jnp.concatenate([hp_ref[i] for i in range(4)], axis=1)
    z = jnp.dot(agg, wl_ref[...], preferred_element_type=jnp.float32)
    z = z + jnp.dot(h1, wr_ref[...], preferred_element_type=jnp.float32)
    z = z + b_ref[...]
    o_ref[...] = (jnp.dot(z, wc_ref[...], preferred_element_type=jnp.float32)
                  + bc_ref[...])


_tc1 = pl.pallas_call(
    _tc1_body,
    grid=(N // R,),
    in_specs=[
        pl.BlockSpec((NC, R, DH), lambda i: (0, i, 0)),
        pl.BlockSpec((R, 16), lambda i: (i, 0)),
        pl.BlockSpec((R, D_IN), lambda i: (i, 0)),
        pl.BlockSpec((D_IN, D_H), lambda i: (0, 0)),
        pl.BlockSpec((D_IN, D_H), lambda i: (0, 0)),
        pl.BlockSpec((1, D_H), lambda i: (0, 0)),
    ],
    out_specs=pl.BlockSpec((4, R, DH), lambda i: (0, i, 0)),
    out_shape=jax.ShapeDtypeStruct((4, N, DH), jnp.float32),
)

_tc2 = pl.pallas_call(
    _tc2_body,
    grid=(N // R,),
    in_specs=[
        pl.BlockSpec((4, R, DH), lambda i: (0, i, 0)),
        pl.BlockSpec((R, 16), lambda i: (i, 0)),
        pl.BlockSpec((4, R, DH), lambda i: (0, i, 0)),
        pl.BlockSpec((D_H, D_H), lambda i: (0, 0)),
        pl.BlockSpec((D_H, D_H), lambda i: (0, 0)),
        pl.BlockSpec((1, D_H), lambda i: (0, 0)),
        pl.BlockSpec((D_H, N_CLASSES), lambda i: (0, 0)),
        pl.BlockSpec((1, N_CLASSES), lambda i: (0, 0)),
    ],
    out_specs=pl.BlockSpec((R, N_CLASSES), lambda i: (i, 0)),
    out_shape=jax.ShapeDtypeStruct((N, N_CLASSES), jnp.float32),
)


def kernel(x, edge_index, Wl1, Wr1, b1, Wl2, Wr2, b2, Wc, bc):
    src = edge_index[0].astype(jnp.int32).reshape(NS, N_CHUNKS, C)
    dst = edge_index[1].astype(jnp.int32).reshape(NS, N_CHUNKS, C)
    x_lo = x[:, :DH]
    x_hi = x[:, DH:]
    summed1, deg16 = _sc_agg_l1()(x_lo, x_hi, src, dst)
    h1q = _tc1(summed1, deg16, x, Wl1, Wr1, b1.reshape(1, -1))
    (summed2,) = _sc_agg_l2()(h1q[0], h1q[1], h1q[2], h1q[3], src, dst)
    out = _tc2(summed2, deg16, h1q, Wl2, Wr2, b2.reshape(1, -1), Wc,
               bc.reshape(1, -1))
    return out


# speedup vs baseline: 10.9860x; 1.0087x over previous
import functools

import jax
import jax.numpy as jnp
from jax import lax
from jax.experimental import pallas as pl
from jax.experimental.pallas import tpu as pltpu
from jax.experimental.pallas import tpu_sc as plsc

N = 10000
E = 320000
D_IN = 128
D_H = 256
N_CLASSES = 64

NC = 2
NS = 16
DH = 64
C = 80
PER_TILE = E // NS
N_CHUNKS = PER_TILE // C
SLAB = 640
SLAB_LAST = N - SLAB * (NS - 1)
ZR = 40
NBUF = 5


def _make_sc_agg(n_phases, with_deg):
    mesh = plsc.VectorSubcoreMesh(core_axis_name="c", subcore_axis_name="s",
                                  num_cores=NC, num_subcores=NS)
    n_t = NC * n_phases
    out_type = [jax.ShapeDtypeStruct((n_t, N, DH), jnp.float32)]
    if with_deg:
        out_type.append(jax.ShapeDtypeStruct((N, 16), jnp.float32))
    scratch = [
        pltpu.VMEM((N_CHUNKS, C), jnp.int32),
        pltpu.VMEM((N_CHUNKS, C), jnp.int32),
        pltpu.VMEM((NBUF, C, DH), jnp.float32),
        pltpu.VMEM((ZR, DH), jnp.float32),
        pltpu.VMEM_SHARED((N, DH), jnp.float32),
        pltpu.SemaphoreType.DMA((NBUF,)),
        pltpu.SemaphoreType.DMA((NBUF,)),
    ]
    if with_deg:
        scratch += [
            pltpu.VMEM((C, 16), jnp.float32),
            pltpu.VMEM((ZR, 16), jnp.float32),
            pltpu.VMEM_SHARED((N, 16), jnp.float32),
            pltpu.SemaphoreType.DMA((NBUF,)),
        ]

    def body(*args):
        tables, src_hbm, dst_hbm, out_hbm = args[:4]
        rest = args[4:]
        if with_deg:
            (deg_hbm, idx_s, idx_d, rows, zbuf, accum, gsem, ssem,
             ones, zbuf16, dacc, dsem) = rest
        else:
            idx_s, idx_d, rows, zbuf, accum, gsem, ssem = rest
        c = lax.axis_index("c")
        s = lax.axis_index("s")
        row0 = s * SLAB

        def for_slab(fn):
            @pl.when(s < NS - 1)
            def _():
                fn(SLAB)

            @pl.when(s == NS - 1)
            def _():
                fn(SLAB_LAST)

        @pl.loop(0, ZR)
        def _(i):
            for k in range(DH // 16):
                zbuf[i, pl.ds(k * 16, 16)] = jnp.zeros((16,), jnp.float32)

        if with_deg:
            @pl.when(c == 0)
            def _():
                @pl.loop(0, C)
                def _(i):
                    ones[i, :] = jnp.full((16,), 1.0, jnp.float32)

                @pl.loop(0, ZR)
                def _(i):
                    zbuf16[i, :] = jnp.zeros((16,), jnp.float32)

                def zero_deg(nr):
                    @pl.loop(0, nr // ZR)
                    def _(i):
                        pltpu.sync_copy(zbuf16,
                                        dacc.at[pl.ds(row0 + i * ZR, ZR)])

                for_slab(zero_deg)

        pltpu.sync_copy(src_hbm.at[s], idx_s)
        pltpu.sync_copy(dst_hbm.at[s], idx_d)

        for p in range(n_phases):
            def zero_accum(nr):
                @pl.loop(0, nr // ZR)
                def _(i):
                    pltpu.sync_copy(zbuf, accum.at[pl.ds(row0 + i * ZR, ZR)])

            for_slab(zero_accum)
            plsc.subcore_barrier()

            def run(table, do_deg):
                for b in range(NBUF):
                    pltpu.async_copy(table.at[idx_s.at[b]], rows.at[b],
                                     gsem.at[b])

                @pl.loop(0, N_CHUNKS, step=NBUF)
                def _(j):
                    for b in range(NBUF):
                        jj = j + b
                        pltpu.make_async_copy(table.at[idx_s.at[jj]],
                                              rows.at[b], gsem.at[b]).wait()
                        pltpu.async_copy(rows.at[b], accum.at[idx_d.at[jj]],
                                         ssem.at[b], add=True)
                        if do_deg:
                            @pl.when(jj >= NBUF)
                            def _():
                                pltpu.make_async_copy(
                                    ones, dacc.at[idx_d.at[jj - NBUF]],
                                    dsem.at[b]).wait()

                            pltpu.async_copy(ones, dacc.at[idx_d.at[jj]],
                                             dsem.at[b], add=True)
                        @pl.when(jj + NBUF < N_CHUNKS)
                        def _():
                            pltpu.make_async_copy(
                                rows.at[b], accum.at[idx_d.at[jj]],
                                ssem.at[b]).wait()
                            pltpu.async_copy(
                                table.at[idx_s.at[jj + NBUF]], rows.at[b],
                                gsem.at[b])

                for b in range(NBUF):
                    jj = N_CHUNKS - NBUF + b
                    pltpu.make_async_copy(rows.at[b], accum.at[idx_d.at[jj]],
                                          ssem.at[b]).wait()
                    if do_deg:
                        pltpu.make_async_copy(ones, dacc.at[idx_d.at[jj]],
                                              dsem.at[b]).wait()

            @pl.when(c == 0)
            def _():
                run(tables.at[2 * p], with_deg and p == 0)

            @pl.when(c == 1)
            def _():
                run(tables.at[2 * p + 1], False)

            plsc.subcore_barrier()

            def copy_out(nr):
                pltpu.sync_copy(accum.at[pl.ds(row0, nr)],
                                out_hbm.at[2 * p + c, pl.ds(row0, nr)])
                if with_deg and p == 0:
                    @pl.when(c == 0)
                    def _():
                        pltpu.sync_copy(dacc.at[pl.ds(row0, nr)],
                                        deg_hbm.at[pl.ds(row0, nr)])

            for_slab(copy_out)
            if p + 1 < n_phases:
                plsc.subcore_barrier()

    return pl.kernel(body, out_type=tuple(out_type), mesh=mesh,
                     scratch_types=scratch,
                     compiler_params=pltpu.CompilerParams(
                         use_tc_tiling_on_sc=False))


_sc_agg_l1 = functools.cache(lambda: _make_sc_agg(1, with_deg=True))
_sc_agg_l2 = functools.cache(lambda: _make_sc_agg(2, with_deg=False))

R = 1000


def _mm_body(x_ref, w_ref, o_ref):
    o_ref[...] = jnp.dot(x_ref[...], w_ref[...],
                         preferred_element_type=jnp.float32)


def _tc1_body(s_ref, deg_ref, xr_ref, wl_ref, b_ref, o_ref):
    deg = jnp.maximum(deg_ref[:, 0:1], 1.0)
    agg = jnp.concatenate([s_ref[0], s_ref[1]], axis=1) / deg
    h = jnp.dot(agg, wl_ref[...], preferred_element_type=jnp.float32)
    h = jnp.maximum(h + xr_ref[...] + b_ref[...], 0.0)
    for q in range(4):
        o_ref[q] = h[:, q * DH:(q + 1) * DH]


def _tcb_body(hp_ref, wr_ref, b_ref, wc_ref, bc_ref, o_ref):
    wrc = jnp.dot(wr_ref[...], wc_ref[...], preferred_element_type=jnp.float32)
    bf = (jnp.dot(b_ref[...], wc_ref[...], preferred_element_type=jnp.float32)
          + bc_ref[...])
    h1 = jnp.concatenate([hp_ref[i] for i in range(4)], axis=1)
    o_ref[...] = jnp.dot(h1, wrc, preferred_element_type=jnp.float32) + bf


def _tc2_body(s_ref, deg_ref, yr_ref, wl_ref, wc_ref, o_ref):
    wlc = jnp.dot(wl_ref[...], wc_ref[...], preferred_element_type=jnp.float32)
    deg = jnp.maximum(deg_ref[:, 0:1], 1.0)
    agg = jnp.concatenate([s_ref[i] for i in range(4)], axis=1) / deg
    o_ref[...] = (jnp.dot(agg, wlc, preferred_element_type=jnp.float32)
                  + yr_ref[...])


_full = pl.BlockSpec(lambda i: (0, 0))

_tc_xr = pl.pallas_call(
    _mm_body,
    grid=(N // R,),
    in_specs=[
        pl.BlockSpec((R, D_IN), lambda i: (i, 0)),
        pl.BlockSpec((D_IN, D_H), lambda i: (0, 0)),
    ],
    out_specs=pl.BlockSpec((R, D_H), lambda i: (i, 0)),
    out_shape=jax.ShapeDtypeStruct((N, D_H), jnp.float32),
)

_tc1 = pl.pallas_call(
    _tc1_body,
    grid=(N // R,),
    in_specs=[
        pl.BlockSpec((NC, R, DH), lambda i: (0, i, 0)),
        pl.BlockSpec((R, 16), lambda i: (i, 0)),
        pl.BlockSpec((R, D_H), lambda i: (i, 0)),
        pl.BlockSpec((D_IN, D_H), lambda i: (0, 0)),
        pl.BlockSpec((1, D_H), lambda i: (0, 0)),
    ],
    out_specs=pl.BlockSpec((4, R, DH), lambda i: (0, i, 0)),
    out_shape=jax.ShapeDtypeStruct((4, N, DH), jnp.float32),
)

_tcb = pl.pallas_call(
    _tcb_body,
    grid=(N // R,),
    in_specs=[
        pl.BlockSpec((4, R, DH), lambda i: (0, i, 0)),
        pl.BlockSpec((D_H, D_H), lambda i: (0, 0)),
        pl.BlockSpec((1, D_H), lambda i: (0, 0)),
        pl.BlockSpec((D_H, N_CLASSES), lambda i: (0, 0)),
        pl.BlockSpec((1, N_CLASSES), lambda i: (0, 0)),
    ],
    out_specs=pl.BlockSpec((R, N_CLASSES), lambda i: (i, 0)),
    out_shape=jax.ShapeDtypeStruct((N, N_CLASSES), jnp.float32),
)

_tc2 = pl.pallas_call(
    _tc2_body,
    grid=(N // R,),
    in_specs=[
        pl.BlockSpec((4, R, DH), lambda i: (0, i, 0)),
        pl.BlockSpec((R, 16), lambda i: (i, 0)),
        pl.BlockSpec((R, N_CLASSES), lambda i: (i, 0)),
        pl.BlockSpec((D_H, D_H), lambda i: (0, 0)),
        pl.BlockSpec((D_H, N_CLASSES), lambda i: (0, 0)),
    ],
    out_specs=pl.BlockSpec((R, N_CLASSES), lambda i: (i, 0)),
    out_shape=jax.ShapeDtypeStruct((N, N_CLASSES), jnp.float32),
)


def kernel(x, edge_index, Wl1, Wr1, b1, Wl2, Wr2, b2, Wc, bc):
    src = edge_index[0].astype(jnp.int32).reshape(NS, N_CHUNKS, C)
    dst = edge_index[1].astype(jnp.int32).reshape(NS, N_CHUNKS, C)
    xq = x.reshape(N, 2, DH).transpose(1, 0, 2)
    summed1, deg16 = _sc_agg_l1()(xq, src, dst)
    xr = _tc_xr(x, Wr1)
    h1q = _tc1(summed1, deg16, xr, Wl1, b1.reshape(1, -1))
    (summed2,) = _sc_agg_l2()(h1q, src, dst)
    yr = _tcb(h1q, Wr2, b2.reshape(1, -1), Wc, bc.reshape(1, -1))
    out = _tc2(summed2, deg16, yr, Wl2, Wc)
    return out


# speedup vs baseline: 10.9939x; 1.0007x over previous
import functools

import jax
import jax.numpy as jnp
from jax import lax
from jax.experimental import pallas as pl
from jax.experimental.pallas import tpu as pltpu
from jax.experimental.pallas import tpu_sc as plsc

N = 10000
E = 320000
D_IN = 128
D_H = 256
N_CLASSES = 64

NC = 2
NS = 16
DH = 64
C = 80
PER_TILE = E // NS
N_CHUNKS = PER_TILE // C
SLAB = 640
SLAB_LAST = N - SLAB * (NS - 1)
ZR = 40
NBUF = 5


def _make_sc_agg(n_phases, with_deg):
    mesh = plsc.VectorSubcoreMesh(core_axis_name="c", subcore_axis_name="s",
                                  num_cores=NC, num_subcores=NS)
    n_t = NC * n_phases
    out_type = [jax.ShapeDtypeStruct((n_t, N, DH), jnp.float32)]
    if with_deg:
        out_type.append(jax.ShapeDtypeStruct((N, 16), jnp.float32))
    scratch = [
        pltpu.VMEM((N_CHUNKS, C), jnp.int32),
        pltpu.VMEM((N_CHUNKS, C), jnp.int32),
        pltpu.VMEM((NBUF, C, DH), jnp.float32),
        pltpu.VMEM((ZR, DH), jnp.float32),
        pltpu.VMEM_SHARED((N, DH), jnp.float32),
        pltpu.SemaphoreType.DMA((NBUF,)),
        pltpu.SemaphoreType.DMA((NBUF,)),
    ]
    if with_deg:
        scratch += [
            pltpu.VMEM((C, 16), jnp.float32),
            pltpu.VMEM((ZR, 16), jnp.float32),
            pltpu.VMEM_SHARED((N, 16), jnp.float32),
            pltpu.SemaphoreType.DMA((NBUF,)),
        ]

    def body(*args):
        tables, src_hbm, dst_hbm, out_hbm = args[:4]
        rest = args[4:]
        if with_deg:
            (deg_hbm, idx_s, idx_d, rows, zbuf, accum, gsem, ssem,
             ones, zbuf16, dacc, dsem) = rest
        else:
            idx_s, idx_d, rows, zbuf, accum, gsem, ssem = rest
        c = lax.axis_index("c")
        s = lax.axis_index("s")
        row0 = s * SLAB

        def for_slab(fn):
            @pl.when(s < NS - 1)
            def _():
                fn(SLAB)

            @pl.when(s == NS - 1)
            def _():
                fn(SLAB_LAST)

        @pl.loop(0, ZR)
        def _(i):
            for k in range(DH // 16):
                zbuf[i, pl.ds(k * 16, 16)] = jnp.zeros((16,), jnp.float32)

        if with_deg:
            @pl.when(c == 0)
            def _():
                @pl.loop(0, C)
                def _(i):
                    ones[i, :] = jnp.full((16,), 1.0, jnp.float32)

                @pl.loop(0, ZR)
                def _(i):
                    zbuf16[i, :] = jnp.zeros((16,), jnp.float32)

                def zero_deg(nr):
                    @pl.loop(0, nr // ZR)
                    def _(i):
                        pltpu.sync_copy(zbuf16,
                                        dacc.at[pl.ds(row0 + i * ZR, ZR)])

                for_slab(zero_deg)

        pltpu.sync_copy(src_hbm.at[s], idx_s)
        pltpu.sync_copy(dst_hbm.at[s], idx_d)

        for p in range(n_phases):
            def zero_accum(nr):
                @pl.loop(0, nr // ZR)
                def _(i):
                    pltpu.sync_copy(zbuf, accum.at[pl.ds(row0 + i * ZR, ZR)])

            for_slab(zero_accum)
            plsc.subcore_barrier()

            def run(table, do_deg):
                for b in range(NBUF):
                    pltpu.async_copy(table.at[idx_s.at[b]], rows.at[b],
                                     gsem.at[b])

                @pl.loop(0, N_CHUNKS, step=NBUF)
                def _(j):
                    for b in range(NBUF):
                        jj = j + b
                        pltpu.make_async_copy(table.at[idx_s.at[jj]],
                                              rows.at[b], gsem.at[b]).wait()
                        pltpu.async_copy(rows.at[b], accum.at[idx_d.at[jj]],
                                         ssem.at[b], add=True)
                        if do_deg:
                            @pl.when(jj >= NBUF)
                            def _():
                                pltpu.make_async_copy(
                                    ones, dacc.at[idx_d.at[jj - NBUF]],
                                    dsem.at[b]).wait()

                            pltpu.async_copy(ones, dacc.at[idx_d.at[jj]],
                                             dsem.at[b], add=True)
                        @pl.when(jj + NBUF < N_CHUNKS)
                        def _():
                            pltpu.make_async_copy(
                                rows.at[b], accum.at[idx_d.at[jj]],
                                ssem.at[b]).wait()
                            pltpu.async_copy(
                                table.at[idx_s.at[jj + NBUF]], rows.at[b],
                                gsem.at[b])

                for b in range(NBUF):
                    jj = N_CHUNKS - NBUF + b
                    pltpu.make_async_copy(rows.at[b], accum.at[idx_d.at[jj]],
                                          ssem.at[b]).wait()
                    if do_deg:
                        pltpu.make_async_copy(ones, dacc.at[idx_d.at[jj]],
                                              dsem.at[b]).wait()

            @pl.when(c == 0)
            def _():
                run(tables.at[2 * p], with_deg and p == 0)

            @pl.when(c == 1)
            def _():
                run(tables.at[2 * p + 1], False)

            plsc.subcore_barrier()

            def copy_out(nr):
                pltpu.sync_copy(accum.at[pl.ds(row0, nr)],
                                out_hbm.at[2 * p + c, pl.ds(row0, nr)])
                if with_deg and p == 0:
                    @pl.when(c == 0)
                    def _():
                        pltpu.sync_copy(dacc.at[pl.ds(row0, nr)],
                                        deg_hbm.at[pl.ds(row0, nr)])

            for_slab(copy_out)
            if p + 1 < n_phases:
                plsc.subcore_barrier()

    return pl.kernel(body, out_type=tuple(out_type), mesh=mesh,
                     scratch_types=scratch,
                     compiler_params=pltpu.CompilerParams(
                         use_tc_tiling_on_sc=False))


_sc_agg_l1 = functools.cache(lambda: _make_sc_agg(1, with_deg=True))
_sc_agg_l2 = functools.cache(lambda: _make_sc_agg(2, with_deg=False))

R = 1000


def _mm_body(x_ref, w_ref, o_ref):
    o_ref[...] = jnp.dot(x_ref[...], w_ref[...],
                         preferred_element_type=jnp.float32)


def _tc1_body(s_ref, deg_ref, xr_ref, wl_ref, b_ref, o_ref):
    deg = jnp.maximum(deg_ref[:, 0:1], 1.0)
    agg = jnp.concatenate([s_ref[0], s_ref[1]], axis=1) / deg
    h = jnp.dot(agg, wl_ref[...], preferred_element_type=jnp.float32)
    h = jnp.maximum(h + xr_ref[...] + b_ref[...], 0.0)
    for q in range(4):
        o_ref[q] = h[:, q * DH:(q + 1) * DH]


def _tcb_body(hp_ref, wr_ref, b_ref, wc_ref, bc_ref, o_ref):
    wrc = jnp.dot(wr_ref[...], wc_ref[...], preferred_element_type=jnp.float32)
    bf = (jnp.dot(b_ref[...], wc_ref[...], preferred_element_type=jnp.float32)
          + bc_ref[...])
    h1 = jnp.concatenate([hp_ref[i] for i in range(4)], axis=1)
    o_ref[...] = jnp.dot(h1, wrc, preferred_element_type=jnp.float32) + bf


def _tc2_body(s_ref, deg_ref, yr_ref, wl_ref, wc_ref, o_ref):
    wlc = jnp.dot(wl_ref[...], wc_ref[...], preferred_element_type=jnp.float32)
    deg = jnp.maximum(deg_ref[:, 0:1], 1.0)
    agg = jnp.concatenate([s_ref[i] for i in range(4)], axis=1) / deg
    o_ref[...] = (jnp.dot(agg, wlc, preferred_element_type=jnp.float32)
                  + yr_ref[...])


_tc_xr = pl.pallas_call(
    _mm_body,
    grid=(N // R,),
    in_specs=[
        pl.BlockSpec((R, D_IN), lambda i: (i, 0)),
        pl.BlockSpec((D_IN, D_H), lambda i: (0, 0)),
    ],
    out_specs=pl.BlockSpec((R, D_H), lambda i: (i, 0)),
    out_shape=jax.ShapeDtypeStruct((N, D_H), jnp.float32),
)

_tc1 = pl.pallas_call(
    _tc1_body,
    grid=(N // R,),
    in_specs=[
        pl.BlockSpec((NC, R, DH), lambda i: (0, i, 0)),
        pl.BlockSpec((R, 16), lambda i: (i, 0)),
        pl.BlockSpec((R, D_H), lambda i: (i, 0)),
        pl.BlockSpec((D_IN, D_H), lambda i: (0, 0)),
        pl.BlockSpec((1, D_H), lambda i: (0, 0)),
    ],
    out_specs=pl.BlockSpec((4, R, DH), lambda i: (0, i, 0)),
    out_shape=jax.ShapeDtypeStruct((4, N, DH), jnp.float32),
)

_tcb = pl.pallas_call(
    _tcb_body,
    grid=(N // R,),
    in_specs=[
        pl.BlockSpec((4, R, DH), lambda i: (0, i, 0)),
        pl.BlockSpec((D_H, D_H), lambda i: (0, 0)),
        pl.BlockSpec((1, D_H), lambda i: (0, 0)),
        pl.BlockSpec((D_H, N_CLASSES), lambda i: (0, 0)),
        pl.BlockSpec((1, N_CLASSES), lambda i: (0, 0)),
    ],
    out_specs=pl.BlockSpec((R, N_CLASSES), lambda i: (i, 0)),
    out_shape=jax.ShapeDtypeStruct((N, N_CLASSES), jnp.float32),
)

_tc2 = pl.pallas_call(
    _tc2_body,
    grid=(N // R,),
    in_specs=[
        pl.BlockSpec((4, R, DH), lambda i: (0, i, 0)),
        pl.BlockSpec((R, 16), lambda i: (i, 0)),
        pl.BlockSpec((R, N_CLASSES), lambda i: (i, 0)),
        pl.BlockSpec((D_H, D_H), lambda i: (0, 0)),
        pl.BlockSpec((D_H, N_CLASSES), lambda i: (0, 0)),
    ],
    out_specs=pl.BlockSpec((R, N_CLASSES), lambda i: (i, 0)),
    out_shape=jax.ShapeDtypeStruct((N, N_CLASSES), jnp.float32),
)


def kernel(x, edge_index, Wl1, Wr1, b1, Wl2, Wr2, b2, Wc, bc):
    src = edge_index[0].astype(jnp.int32).reshape(NS, N_CHUNKS, C)
    dst = edge_index[1].astype(jnp.int32).reshape(NS, N_CHUNKS, C)
    xq = x.reshape(N, 2, DH).transpose(1, 0, 2)
    summed1, deg16 = _sc_agg_l1()(xq, src, dst)
    xr = _tc_xr(x, Wr1)
    h1q = _tc1(summed1, deg16, xr, Wl1, b1.reshape(1, -1))
    (summed2,) = _sc_agg_l2()(h1q, src, dst)
    yr = _tcb(h1q, Wr2, b2.reshape(1, -1), Wc, bc.reshape(1, -1))
    out = _tc2(summed2, deg16, yr, Wl2, Wc)
    return out
